# baseline (speedup 1.0000x reference)
import functools

import jax
import jax.numpy as jnp
from jax import lax
from jax.experimental import pallas as pl
from jax.experimental.pallas import tpu as pltpu
from jax.experimental.pallas import tpu_sc as plsc

_KB = 2048

_NC, _NS = 2, 16
_NW = _NC * _NS


_R = 8


def _vq_argmax_kernel(x_ref, xx_ref, embed2_bf_ref, ee_ref, idx_ref, m_scr):
    xb = x_ref[0]
    xx = xx_ref[0]
    D, N = xb.shape
    K = embed2_bf_ref.shape[0]
    xb_bf = xb.astype(jnp.bfloat16)
    xx8 = jnp.broadcast_to(xx, (_R, N))
    sio = jax.lax.broadcasted_iota(jnp.int32, (_R, N), 0)

    def body(c, carry):
        run_min_bf, run_idx = carry
        e_blk = embed2_bf_ref[pl.ds(c * _KB, _KB), :]
        m_scr[...] = jnp.dot(e_blk, xb_bf,
                             preferred_element_type=jnp.float32)

        def rbody(r, rc):
            min8, arg8 = rc
            mv = m_scr[pl.ds(r * _R, _R), :]
            ee8 = ee_ref[pl.ds(c * _KB + r * _R, _R), :]
            nd = (xx8 - mv) + ee8
            lt = nd < min8
            min8 = jnp.where(lt, nd, min8)
            arg8 = jnp.where(lt, r, arg8)
            return min8, arg8

        min8, arg8 = jax.lax.fori_loop(
            0, _KB // _R, rbody,
            (jnp.full((_R, N), jnp.inf, jnp.float32),
             jnp.zeros((_R, N), jnp.int32)))
        cmin = jnp.min(min8, axis=0, keepdims=True)
        k8 = arg8 * _R + sio
        masked = jnp.where(min8 == cmin, k8, _KB)
        lidx = jnp.min(masked, axis=0, keepdims=True)
        better = cmin < run_min_bf.astype(jnp.float32)
        run_min_bf = jnp.where(better, cmin,
                               run_min_bf.astype(jnp.float32)).astype(jnp.bfloat16)
        run_idx = jnp.where(better, lidx + c * _KB, run_idx)
        return run_min_bf, run_idx

    init = (jnp.full((1, N), jnp.inf, jnp.bfloat16),
            jnp.zeros((1, N), jnp.int32))
    _, run_idx = jax.lax.fori_loop(0, K // _KB, body, init)
    idx_ref[0] = run_idx


def _sc_gather(table, idx):
    V, D = table.shape
    (B_tok,) = idx.shape
    b_per_w = B_tok // _NW

    @functools.partial(
        pl.kernel,
        mesh=plsc.VectorSubcoreMesh(core_axis_name="c", subcore_axis_name="s"),
        out_type=jax.ShapeDtypeStruct((B_tok, D), jnp.float32),
        scratch_types=[
            pltpu.VMEM((b_per_w,), jnp.int32),
            pltpu.VMEM((b_per_w, D), jnp.float32),
            pltpu.SemaphoreType.DMA,
        ],
    )
    def gather_kernel(table_hbm, idx_hbm, out_hbm, idx_v, rows_v, sem):
        wid = lax.axis_index("s") * _NC + lax.axis_index("c")
        base = wid * b_per_w
        pltpu.sync_copy(idx_hbm.at[pl.ds(base, b_per_w)], idx_v)
        pltpu.async_copy(table_hbm.at[idx_v], rows_v, sem).wait()
        pltpu.sync_copy(rows_v, out_hbm.at[pl.ds(base, b_per_w)])

    return gather_kernel(table, idx)


def kernel(x, embed):
    B, D, N = x.shape
    K = embed.shape[0]
    xt = jnp.transpose(x, (0, 2, 1)).reshape(-1, D)
    embed_t = embed.T
    xx = jnp.sum(xt * xt, axis=1, keepdims=True).reshape(B, 1, N)
    ee = jnp.sum(embed_t * embed_t, axis=0, keepdims=True).reshape(K, 1)
    embed2_bf = (2.0 * embed.astype(jnp.bfloat16).astype(jnp.float32)
                 ).astype(jnp.bfloat16)

    idx = pl.pallas_call(
        _vq_argmax_kernel,
        grid=(B,),
        in_specs=[
            pl.BlockSpec((1, D, N), lambda b: (b, 0, 0)),
            pl.BlockSpec((1, 1, N), lambda b: (b, 0, 0)),
            pl.BlockSpec((K, D), lambda b: (0, 0)),
            pl.BlockSpec((K, 1), lambda b: (0, 0)),
        ],
        out_specs=pl.BlockSpec((1, 1, N), lambda b: (b, 0, 0)),
        out_shape=jax.ShapeDtypeStruct((B, 1, N), jnp.int32),
        scratch_shapes=[pltpu.VMEM((_KB, N), jnp.float32)],
    )(x, xx, embed2_bf, ee)

    idx_flat = idx.reshape(-1)
    table128 = jnp.pad(embed, ((0, 0), (0, 128 - D)))
    rows128 = _sc_gather(table128, idx_flat)
    quantize = jnp.transpose(rows128.reshape(B, N, 128)[:, :, :D], (0, 2, 1))
    return quantize, idx.reshape(B, N)


# speedup vs baseline: 3.5244x; 3.5244x over previous
import functools

import jax
import jax.numpy as jnp
from jax import lax
from jax.experimental import pallas as pl
from jax.experimental.pallas import tpu as pltpu
from jax.experimental.pallas import tpu_sc as plsc

_KB = 2048

_NC, _NS = 2, 16
_NW = _NC * _NS


_R = 8
_U = 8


def _vq_argmax_kernel(x_ref, xx_ref, embed2_bf_ref, ee_ref, idx_ref, m_scr):
    xb = x_ref[0]
    xx = xx_ref[0]
    D, N = xb.shape
    K = embed2_bf_ref.shape[0]
    xb_bf = xb.astype(jnp.bfloat16)
    xx8 = jnp.broadcast_to(xx, (_R, N))
    sio = jax.lax.broadcasted_iota(jnp.int32, (_R, N), 0)

    def body(c, carry):
        run_min_bf, run_idx = carry
        e_blk = embed2_bf_ref[pl.ds(c * _KB, _KB), :]
        m_scr[...] = jnp.dot(e_blk, xb_bf,
                             preferred_element_type=jnp.float32)

        def rbody(i, rc):
            min8, arg8 = rc
            for j in range(_U):
                r = i * _U + j
                mv = m_scr[pl.ds(r * _R, _R), :]
                ee8 = ee_ref[pl.ds(c * _KB + r * _R, _R), :]
                nd = (xx8 - mv) + ee8
                lt = nd < min8
                min8 = jnp.where(lt, nd, min8)
                arg8 = jnp.where(lt, r, arg8)
            return min8, arg8

        min8, arg8 = jax.lax.fori_loop(
            0, _KB // (_R * _U), rbody,
            (jnp.full((_R, N), jnp.inf, jnp.float32),
             jnp.zeros((_R, N), jnp.int32)))
        cmin = jnp.min(min8, axis=0, keepdims=True)
        k8 = arg8 * _R + sio
        masked = jnp.where(min8 == cmin, k8, _KB)
        lidx = jnp.min(masked, axis=0, keepdims=True)
        better = cmin < run_min_bf.astype(jnp.float32)
        run_min_bf = jnp.where(better, cmin,
                               run_min_bf.astype(jnp.float32)).astype(jnp.bfloat16)
        run_idx = jnp.where(better, lidx + c * _KB, run_idx)
        return run_min_bf, run_idx

    init = (jnp.full((1, N), jnp.inf, jnp.bfloat16),
            jnp.zeros((1, N), jnp.int32))
    _, run_idx = jax.lax.fori_loop(0, K // _KB, body, init)
    idx_ref[0] = run_idx


def _sc_gather(table, idx):
    V, D = table.shape
    (B_tok,) = idx.shape
    b_per_w = B_tok // _NW

    @functools.partial(
        pl.kernel,
        mesh=plsc.VectorSubcoreMesh(core_axis_name="c", subcore_axis_name="s"),
        out_type=jax.ShapeDtypeStruct((B_tok, D), jnp.float32),
        scratch_types=[
            pltpu.VMEM((b_per_w,), jnp.int32),
            pltpu.VMEM((b_per_w, D), jnp.float32),
            pltpu.SemaphoreType.DMA,
        ],
    )
    def gather_kernel(table_hbm, idx_hbm, out_hbm, idx_v, rows_v, sem):
        wid = lax.axis_index("s") * _NC + lax.axis_index("c")
        base = wid * b_per_w
        pltpu.sync_copy(idx_hbm.at[pl.ds(base, b_per_w)], idx_v)
        pltpu.async_copy(table_hbm.at[idx_v], rows_v, sem).wait()
        pltpu.sync_copy(rows_v, out_hbm.at[pl.ds(base, b_per_w)])

    return gather_kernel(table, idx)


def kernel(x, embed):
    B, D, N = x.shape
    K = embed.shape[0]
    xt = jnp.transpose(x, (0, 2, 1)).reshape(-1, D)
    embed_t = embed.T
    xx = jnp.sum(xt * xt, axis=1, keepdims=True).reshape(B, 1, N)
    ee = jnp.sum(embed_t * embed_t, axis=0, keepdims=True).reshape(K, 1)
    embed2_bf = (2.0 * embed.astype(jnp.bfloat16).astype(jnp.float32)
                 ).astype(jnp.bfloat16)

    idx = pl.pallas_call(
        _vq_argmax_kernel,
        grid=(B,),
        in_specs=[
            pl.BlockSpec((1, D, N), lambda b: (b, 0, 0)),
            pl.BlockSpec((1, 1, N), lambda b: (b, 0, 0)),
            pl.BlockSpec((K, D), lambda b: (0, 0)),
            pl.BlockSpec((K, 1), lambda b: (0, 0)),
        ],
        out_specs=pl.BlockSpec((1, 1, N), lambda b: (b, 0, 0)),
        out_shape=jax.ShapeDtypeStruct((B, 1, N), jnp.int32),
        scratch_shapes=[pltpu.VMEM((_KB, N), jnp.float32)],
    )(x, xx, embed2_bf, ee)

    idx_flat = idx.reshape(-1)
    table128 = jnp.pad(embed, ((0, 0), (0, 128 - D)))
    rows128 = _sc_gather(table128, idx_flat)
    quantize = jnp.transpose(rows128.reshape(B, N, 128)[:, :, :D], (0, 2, 1))
    return quantize, idx.reshape(B, N)


# speedup vs baseline: 6.0554x; 1.7181x over previous
import functools

import jax
import jax.numpy as jnp
from jax import lax
from jax.experimental import pallas as pl
from jax.experimental.pallas import tpu as pltpu
from jax.experimental.pallas import tpu_sc as plsc

_KB = 2048

_NC, _NS = 2, 16
_NW = _NC * _NS


def _vq_argmax_kernel(x_ref, xx_ref, embed2_bf_ref, ee_ref, idx_ref):
    xb = x_ref[0]
    xx = xx_ref[0]
    D, N = xb.shape
    K = embed2_bf_ref.shape[0]
    xb_bf = xb.astype(jnp.bfloat16)
    kio_f = jax.lax.broadcasted_iota(jnp.int32, (_KB, N), 0).astype(jnp.float32)

    run_min_bf = jnp.full((1, N), jnp.inf, jnp.bfloat16)
    run_idx = jnp.zeros((1, N), jnp.int32)
    for c in range(K // _KB):
        e_blk = embed2_bf_ref[pl.ds(c * _KB, _KB), :]
        ee = ee_ref[pl.ds(c * _KB, _KB), :]
        m2 = jnp.dot(e_blk, xb_bf,
                     preferred_element_type=jnp.float32)
        nd = (xx - m2) + ee
        cmin = jnp.min(nd, axis=0, keepdims=True)
        masked = jnp.where(nd == cmin, kio_f, float(_KB))
        lidx_f = jnp.min(masked, axis=0, keepdims=True)
        lidx = lidx_f.astype(jnp.int32)
        better = cmin < run_min_bf.astype(jnp.float32)
        run_min_bf = jnp.where(better, cmin,
                               run_min_bf.astype(jnp.float32)).astype(jnp.bfloat16)
        run_idx = jnp.where(better, lidx + c * _KB, run_idx)

    idx_ref[0] = run_idx


def _sc_gather(table, idx):
    V, D = table.shape
    (B_tok,) = idx.shape
    b_per_w = B_tok // _NW

    @functools.partial(
        pl.kernel,
        mesh=plsc.VectorSubcoreMesh(core_axis_name="c", subcore_axis_name="s"),
        out_type=jax.ShapeDtypeStruct((B_tok, D), jnp.float32),
        scratch_types=[
            pltpu.VMEM((b_per_w,), jnp.int32),
            pltpu.VMEM((b_per_w, D), jnp.float32),
            pltpu.SemaphoreType.DMA,
        ],
    )
    def gather_kernel(table_hbm, idx_hbm, out_hbm, idx_v, rows_v, sem):
        wid = lax.axis_index("s") * _NC + lax.axis_index("c")
        base = wid * b_per_w
        pltpu.sync_copy(idx_hbm.at[pl.ds(base, b_per_w)], idx_v)
        pltpu.async_copy(table_hbm.at[idx_v], rows_v, sem).wait()
        pltpu.sync_copy(rows_v, out_hbm.at[pl.ds(base, b_per_w)])

    return gather_kernel(table, idx)


def kernel(x, embed):
    B, D, N = x.shape
    K = embed.shape[0]
    xt = jnp.transpose(x, (0, 2, 1)).reshape(-1, D)
    embed_t = embed.T
    xx = jnp.sum(xt * xt, axis=1, keepdims=True).reshape(B, 1, N)
    ee = jnp.sum(embed_t * embed_t, axis=0, keepdims=True).reshape(K, 1)
    embed2_bf = (2.0 * embed.astype(jnp.bfloat16).astype(jnp.float32)
                 ).astype(jnp.bfloat16)

    idx = pl.pallas_call(
        _vq_argmax_kernel,
        grid=(B,),
        in_specs=[
            pl.BlockSpec((1, D, N), lambda b: (b, 0, 0)),
            pl.BlockSpec((1, 1, N), lambda b: (b, 0, 0)),
            pl.BlockSpec((K, D), lambda b: (0, 0)),
            pl.BlockSpec((K, 1), lambda b: (0, 0)),
        ],
        out_specs=pl.BlockSpec((1, 1, N), lambda b: (b, 0, 0)),
        out_shape=jax.ShapeDtypeStruct((B, 1, N), jnp.int32),
    )(x, xx, embed2_bf, ee)

    idx_flat = idx.reshape(-1)
    table128 = jnp.pad(embed, ((0, 0), (0, 128 - D)))
    rows128 = _sc_gather(table128, idx_flat)
    quantize = jnp.transpose(rows128.reshape(B, N, 128)[:, :, :D], (0, 2, 1))
    return quantize, idx.reshape(B, N)


# speedup vs baseline: 6.1516x; 1.0159x over previous
import functools

import jax
import jax.numpy as jnp
from jax import lax
from jax.experimental import pallas as pl
from jax.experimental.pallas import tpu as pltpu
from jax.experimental.pallas import tpu_sc as plsc

_KB = 2048

_NC, _NS = 2, 16
_NW = _NC * _NS


def _vq_argmax_kernel(x_ref, xx_ref, embed2_bf_ref, ee_ref, idx_ref):
    xb = x_ref[0]
    xx = xx_ref[0]
    D, N = xb.shape
    K = embed2_bf_ref.shape[0]
    xb_bf = xb.astype(jnp.bfloat16)
    kio_f = jax.lax.broadcasted_iota(jnp.int32, (_KB, N), 0).astype(jnp.float32)

    run_min_bf = jnp.full((1, N), jnp.inf, jnp.bfloat16)
    run_idx = jnp.zeros((1, N), jnp.int32)
    for c in range(K // _KB):
        e_blk = embed2_bf_ref[pl.ds(c * _KB, _KB), :]
        ee = ee_ref[pl.ds(c * _KB, _KB), :]
        m2 = jnp.dot(e_blk, xb_bf,
                     preferred_element_type=jnp.float32)
        nd = (xx - m2) + ee
        h = _KB // 2
        cmin = jnp.minimum(
            jnp.min(nd[:h], axis=0, keepdims=True),
            jnp.min(nd[h:], axis=0, keepdims=True))
        masked = jnp.where(nd == cmin, kio_f, float(_KB))
        lidx_f = jnp.minimum(
            jnp.min(masked[:h], axis=0, keepdims=True),
            jnp.min(masked[h:], axis=0, keepdims=True))
        lidx = lidx_f.astype(jnp.int32)
        better = cmin < run_min_bf.astype(jnp.float32)
        run_min_bf = jnp.where(better, cmin,
                               run_min_bf.astype(jnp.float32)).astype(jnp.bfloat16)
        run_idx = jnp.where(better, lidx + c * _KB, run_idx)

    idx_ref[0] = run_idx


def _sc_gather(table, idx):
    V, D = table.shape
    (B_tok,) = idx.shape
    b_per_w = B_tok // _NW

    @functools.partial(
        pl.kernel,
        mesh=plsc.VectorSubcoreMesh(core_axis_name="c", subcore_axis_name="s"),
        out_type=jax.ShapeDtypeStruct((B_tok, D), jnp.float32),
        scratch_types=[
            pltpu.VMEM((b_per_w,), jnp.int32),
            pltpu.VMEM((b_per_w, D), jnp.float32),
            pltpu.SemaphoreType.DMA,
        ],
    )
    def gather_kernel(table_hbm, idx_hbm, out_hbm, idx_v, rows_v, sem):
        wid = lax.axis_index("s") * _NC + lax.axis_index("c")
        base = wid * b_per_w
        pltpu.sync_copy(idx_hbm.at[pl.ds(base, b_per_w)], idx_v)
        pltpu.async_copy(table_hbm.at[idx_v], rows_v, sem).wait()
        pltpu.sync_copy(rows_v, out_hbm.at[pl.ds(base, b_per_w)])

    return gather_kernel(table, idx)


def kernel(x, embed):
    B, D, N = x.shape
    K = embed.shape[0]
    xt = jnp.transpose(x, (0, 2, 1)).reshape(-1, D)
    embed_t = embed.T
    xx = jnp.sum(xt * xt, axis=1, keepdims=True).reshape(B, 1, N)
    ee = jnp.sum(embed_t * embed_t, axis=0, keepdims=True).reshape(K, 1)
    embed2_bf = (2.0 * embed.astype(jnp.bfloat16).astype(jnp.float32)
                 ).astype(jnp.bfloat16)

    idx = pl.pallas_call(
        _vq_argmax_kernel,
        grid=(B,),
        in_specs=[
            pl.BlockSpec((1, D, N), lambda b: (b, 0, 0)),
            pl.BlockSpec((1, 1, N), lambda b: (b, 0, 0)),
            pl.BlockSpec((K, D), lambda b: (0, 0)),
            pl.BlockSpec((K, 1), lambda b: (0, 0)),
        ],
        out_specs=pl.BlockSpec((1, 1, N), lambda b: (b, 0, 0)),
        out_shape=jax.ShapeDtypeStruct((B, 1, N), jnp.int32),
    )(x, xx, embed2_bf, ee)

    idx_flat = idx.reshape(-1)
    table128 = jnp.pad(embed, ((0, 0), (0, 128 - D)))
    rows128 = _sc_gather(table128, idx_flat)
    quantize = jnp.transpose(rows128.reshape(B, N, 128)[:, :, :D], (0, 2, 1))
    return quantize, idx.reshape(B, N)
